# Initial kernel scaffold; baseline (speedup 1.0000x reference)
import jax
import jax.numpy as jnp
from jax.experimental import pallas as pl


def kernel(x, edge_index, batch, W0, b0, W1, b1, W2, b2, Wl, bl):
    raise NotImplementedError("write your pallas kernel here")


# speedup vs baseline: 13.2459x; 13.2459x over previous
import functools

import jax
import jax.numpy as jnp
from jax import lax
from jax.experimental import pallas as pl
from jax.experimental.pallas import tpu as pltpu
from jax.experimental.pallas import tpu_sc as plsc

N = 10000
NPAD = 10240
E = 320000
D = 128
H = 128
C = 64
G = 128

NC = 2
NS = 16
NW = NC * NS
EG = 128
E_ALL = E + N
GPT = -(-E_ALL // (NW * EG))
E_PAD = NW * EG * GPT
ROWS_PER_TILE = NPAD // NW
ROWS_PER_SUB = NPAD // NS


def _zero_vmem_rows(buf, nrows, width):
    z = jnp.zeros((16,), jnp.float32)

    def body(i, _):
        for j in range(width // 16):
            buf[i, pl.ds(j * 16, 16)] = z
        return 0

    lax.fori_loop(0, nrows, body, 0)


def _fill_vmem_rows(buf, nrows, width, value):
    v = jnp.full((16,), value, jnp.float32)

    def body(i, _):
        for j in range(width // 16):
            buf[i, pl.ds(j * 16, 16)] = v
        return 0

    lax.fori_loop(0, nrows, body, 0)


def _deg_body(dst_hbm, out_hbm, dst_v, ones_v, acc_sh):
    c = lax.axis_index("c")
    s = lax.axis_index("s")
    wid = s * NC + c

    _zero_vmem_rows(ones_v, EG, H)
    for g in range(ROWS_PER_SUB // EG):
        pltpu.sync_copy(ones_v, acc_sh.at[pl.ds(s * ROWS_PER_SUB + g * EG, EG)])
    _fill_vmem_rows(ones_v, EG, H, 1.0)
    plsc.subcore_barrier()

    pltpu.sync_copy(dst_hbm.at[wid], dst_v)

    def body(j, _):
        pltpu.sync_copy(ones_v, acc_sh.at[dst_v.at[j]], add=True)
        return 0

    lax.fori_loop(0, GPT, body, 0)
    plsc.subcore_barrier()

    pltpu.sync_copy(
        acc_sh.at[pl.ds(s * ROWS_PER_SUB, ROWS_PER_SUB)],
        out_hbm.at[c, pl.ds(s * ROWS_PER_SUB, ROWS_PER_SUB)],
    )


@functools.cache
def _sc_kernels():
    mesh = plsc.VectorSubcoreMesh(core_axis_name="c", subcore_axis_name="s")
    deg = pl.kernel(
        _deg_body,
        out_type=jax.ShapeDtypeStruct((NC, NPAD, H), jnp.float32),
        mesh=mesh,
        scratch_types=[
            pltpu.VMEM((GPT, EG), jnp.int32),
            pltpu.VMEM((EG, H), jnp.float32),
            pltpu.VMEM_SHARED((NPAD, H), jnp.float32),
        ],
    )
    prop = pl.kernel(
        _prop_body,
        out_type=jax.ShapeDtypeStruct((NC, NPAD, H), jnp.float32),
        mesh=mesh,
        scratch_types=[
            pltpu.VMEM((GPT, EG), jnp.int32),
            pltpu.VMEM((GPT, EG), jnp.int32),
            pltpu.VMEM((EG, H), jnp.float32),
            pltpu.VMEM_SHARED((NPAD, H), jnp.float32),
            pltpu.SemaphoreType.DMA,
        ],
    )
    return deg, prop


def _prop_body(table_hbm, src_hbm, dst_hbm, out_hbm, src_v, dst_v, rows_v,
                 acc_sh, sem):
    c = lax.axis_index("c")
    s = lax.axis_index("s")
    wid = s * NC + c

    _zero_vmem_rows(rows_v, EG, H)
    for g in range(ROWS_PER_SUB // EG):
        pltpu.sync_copy(rows_v, acc_sh.at[pl.ds(s * ROWS_PER_SUB + g * EG, EG)])
    plsc.subcore_barrier()

    pltpu.sync_copy(src_hbm.at[wid], src_v)
    pltpu.sync_copy(dst_hbm.at[wid], dst_v)

    def body(j, _):
        pltpu.async_copy(table_hbm.at[src_v.at[j]], rows_v, sem).wait()
        pltpu.sync_copy(rows_v, acc_sh.at[dst_v.at[j]], add=True)
        return 0

    lax.fori_loop(0, GPT, body, 0)
    plsc.subcore_barrier()

    pltpu.sync_copy(
        acc_sh.at[pl.ds(s * ROWS_PER_SUB, ROWS_PER_SUB)],
        out_hbm.at[c, pl.ds(s * ROWS_PER_SUB, ROWS_PER_SUB)],
    )


_BM = 1024
_GRID = NPAD // _BM


def _dinv_block(degb):
    deg = degb[0, :, 0:1] + degb[1, :, 0:1]
    return lax.rsqrt(jnp.maximum(deg, 1.0))


def _tc_first_body(xb, wb, degb, tableb):
    t = jnp.dot(xb[...], wb[...], preferred_element_type=jnp.float32)
    tableb[...] = t * _dinv_block(degb[...])


def _tc_first(x_pad, w, degp):
    return pl.pallas_call(
        _tc_first_body,
        grid=(_GRID,),
        in_specs=[
            pl.BlockSpec((_BM, D), lambda i: (i, 0)),
            pl.BlockSpec((D, H), lambda i: (0, 0)),
            pl.BlockSpec((NC, _BM, H), lambda i: (0, i, 0)),
        ],
        out_specs=pl.BlockSpec((_BM, H), lambda i: (i, 0)),
        out_shape=jax.ShapeDtypeStruct((NPAD, H), jnp.float32),
    )(x_pad, w, degp)


def _tc_mid_body(accb, degb, bb, wb, tableb):
    dinv = _dinv_block(degb[...])
    acc = accb[0] + accb[1]
    h = jnp.maximum(acc * dinv + bb[...], 0.0)
    t = jnp.dot(h, wb[...], preferred_element_type=jnp.float32)
    tableb[...] = t * dinv


def _tc_mid(accp, degp, b_row, w):
    return pl.pallas_call(
        _tc_mid_body,
        grid=(_GRID,),
        in_specs=[
            pl.BlockSpec((NC, _BM, H), lambda i: (0, i, 0)),
            pl.BlockSpec((NC, _BM, H), lambda i: (0, i, 0)),
            pl.BlockSpec((1, H), lambda i: (0, 0)),
            pl.BlockSpec((H, H), lambda i: (0, 0)),
        ],
        out_specs=pl.BlockSpec((_BM, H), lambda i: (i, 0)),
        out_shape=jax.ShapeDtypeStruct((NPAD, H), jnp.float32),
    )(accp, degp, b_row, w)


def _tc_final_body(accb, degb, bb, wlb, blb, batchb, y_out, gm_out,
                   sums_s, cnts_s):
    i = pl.program_id(0)

    @pl.when(i == 0)
    def _():
        sums_s[...] = jnp.zeros_like(sums_s)
        cnts_s[...] = jnp.zeros_like(cnts_s)

    dinv = _dinv_block(degb[...])
    acc = accb[0] + accb[1]
    h = jnp.maximum(acc * dinv + bb[...], 0.0)
    oh = (batchb[...] == lax.broadcasted_iota(jnp.int32, (_BM, G), 1))
    oh = oh.astype(jnp.float32)
    sums_s[...] += lax.dot_general(
        oh, h, (((0,), (0,)), ((), ())), preferred_element_type=jnp.float32)
    cnts_s[...] += lax.dot_general(
        oh, jnp.ones((_BM, 1), jnp.float32), (((0,), (0,)), ((), ())),
        preferred_element_type=jnp.float32)

    @pl.when(i == pl.num_programs(0) - 1)
    def _():
        gm = sums_s[...] / jnp.maximum(cnts_s[...], 1.0)
        gm_out[...] = gm
        y_out[...] = jnp.dot(gm, wlb[...],
                             preferred_element_type=jnp.float32) + blb[...]


def _tc_final(accp, degp, b_row, wl, bl_row, batch2d):
    return pl.pallas_call(
        _tc_final_body,
        grid=(_GRID,),
        in_specs=[
            pl.BlockSpec((NC, _BM, H), lambda i: (0, i, 0)),
            pl.BlockSpec((NC, _BM, H), lambda i: (0, i, 0)),
            pl.BlockSpec((1, H), lambda i: (0, 0)),
            pl.BlockSpec((H, C), lambda i: (0, 0)),
            pl.BlockSpec((1, C), lambda i: (0, 0)),
            pl.BlockSpec((_BM, 1), lambda i: (i, 0)),
        ],
        out_specs=[
            pl.BlockSpec((G, C), lambda i: (0, 0)),
            pl.BlockSpec((G, H), lambda i: (0, 0)),
        ],
        out_shape=[
            jax.ShapeDtypeStruct((G, C), jnp.float32),
            jax.ShapeDtypeStruct((G, H), jnp.float32),
        ],
        scratch_shapes=[
            pltpu.VMEM((G, H), jnp.float32),
            pltpu.VMEM((G, 1), jnp.float32),
        ],
    )(accp, degp, b_row, wl, bl_row, batch2d)


def kernel(x, edge_index, batch, W0, b0, W1, b1, W2, b2, Wl, bl):
    x_pad = jnp.pad(x, ((0, NPAD - N), (0, 0)))
    loops = jnp.arange(N, dtype=jnp.int32)
    padv = jnp.full((E_PAD - E_ALL,), N, jnp.int32)
    src_blk = jnp.concatenate([edge_index[0], loops, padv]).reshape(NW, GPT, EG)
    dst_blk = jnp.concatenate([edge_index[1], loops, padv]).reshape(NW, GPT, EG)
    batch2d = jnp.pad(batch, (0, NPAD - N), constant_values=G).reshape(NPAD, 1)
    b0r = b0.reshape(1, H)
    b1r = b1.reshape(1, H)
    b2r = b2.reshape(1, H)
    blr = bl.reshape(1, C)

    deg_kernel, prop_kernel = _sc_kernels()
    degp = deg_kernel(dst_blk)

    table = _tc_first(x_pad, W0, degp)
    accp = prop_kernel(table, src_blk, dst_blk)
    table = _tc_mid(accp, degp, b0r, W1)
    accp = prop_kernel(table, src_blk, dst_blk)
    table = _tc_mid(accp, degp, b1r, W2)
    accp = prop_kernel(table, src_blk, dst_blk)
    y, gm = _tc_final(accp, degp, b2r, Wl, blr, batch2d)
    return (y, gm)
